# baseline (speedup 1.0000x reference)
import functools

import jax
import jax.numpy as jnp
from jax import lax
from jax.experimental import pallas as pl
from jax.experimental.pallas import tpu as pltpu, tpu_sc as plsc

_LANES = 16
_NCHUNK = 4


def _sc_interp_kernel(rows_per_w, cols, n, x_hbm, y_hbm, out_hbm,
                      y_v, dy_v, ib0, ib1, ob0, ob1, si0, si1, so0, so1):
    wid = lax.axis_index("s") * 2 + lax.axis_index("c")
    row0 = wid * rows_per_w
    crows = rows_per_w // _NCHUNK

    ibufs, obufs = (ib0, ib1), (ob0, ob1)
    isems, osems = (si0, si1), (so0, so1)

    def start_in(k):
        return pltpu.async_copy(
            x_hbm.at[pl.ds(row0 + k * crows, crows)], ibufs[k % 2],
            isems[k % 2])

    def start_out(k):
        return pltpu.async_copy(
            obufs[k % 2], out_hbm.at[pl.ds(row0 + k * crows, crows)],
            osems[k % 2])

    in_cp = {0: start_in(0), 1: start_in(1)}
    pltpu.sync_copy(y_hbm, y_v)

    for j in range((n - 1) // _LANES):
        v = y_v[pl.ds(j * _LANES, _LANES)]
        vn = y_v[pl.ds(j * _LANES + 1, _LANES)]
        dy_v[pl.ds(j * _LANES, _LANES)] = vn - v

    scale = jnp.float32(n - 1)
    offs = list(range(0, cols - _LANES, _LANES)) + [cols - _LANES]
    out_cp = {}

    for k in range(_NCHUNK):
        ib, ob = ibufs[k % 2], obufs[k % 2]
        in_cp.pop(k).wait()
        if k >= 2:
            out_cp.pop(k - 2).wait()

        @plsc.parallel_loop(0, crows, unroll=8)
        def body(r, ib=ib, ob=ob):
            xs = [ib[r, pl.ds(c, _LANES)] for c in offs]
            for c, xv in zip(offs, xs):
                t = xv * scale
                i0 = t.astype(jnp.int32)
                w = t - i0.astype(jnp.float32)
                y0 = plsc.load_gather(y_v, [i0])
                d0 = plsc.load_gather(dy_v, [i0])
                ob[r, pl.ds(c, _LANES)] = y0 + w * d0
        out_cp[k] = start_out(k)
        if k + 2 < _NCHUNK:
            in_cp[k + 2] = start_in(k + 2)

    for k in sorted(out_cp):
        out_cp[k].wait()


def kernel(x, y):
    n = y.shape[0]
    rows, cols = x.shape
    nw = 32
    rows_per_w = rows // nw
    assert rows_per_w * nw == rows and cols >= _LANES
    assert (n - 1) % _LANES == 0 and rows_per_w % _NCHUNK == 0
    crows = rows_per_w // _NCHUNK

    mesh = plsc.VectorSubcoreMesh(core_axis_name="c", subcore_axis_name="s")
    run = pl.kernel(
        functools.partial(_sc_interp_kernel, rows_per_w, cols, n),
        mesh=mesh,
        out_type=jax.ShapeDtypeStruct((rows, cols), jnp.float32),
        scratch_types=[
            pltpu.VMEM((n,), jnp.float32),
            pltpu.VMEM((n - 1,), jnp.float32),
            pltpu.VMEM((crows, cols), jnp.float32),
            pltpu.VMEM((crows, cols), jnp.float32),
            pltpu.VMEM((crows, cols), jnp.float32),
            pltpu.VMEM((crows, cols), jnp.float32),
            pltpu.SemaphoreType.DMA,
            pltpu.SemaphoreType.DMA,
            pltpu.SemaphoreType.DMA,
            pltpu.SemaphoreType.DMA,
        ],
        compiler_params=pltpu.CompilerParams(needs_layout_passes=False),
    )
    return run(x, y)


# speedup vs baseline: 1.0001x; 1.0001x over previous
import functools

import jax
import jax.numpy as jnp
from jax import lax
from jax.experimental import pallas as pl
from jax.experimental.pallas import tpu as pltpu, tpu_sc as plsc

_LANES = 16
_NCHUNK = 4


def _sc_interp_kernel(rows_per_w, cols, n, x_hbm, y_hbm, out_hbm,
                      y_v, dy_v, ib0, ib1, ob0, ob1, si0, si1, so0, so1):
    wid = lax.axis_index("s") * 2 + lax.axis_index("c")
    row0 = wid * rows_per_w
    crows = rows_per_w // _NCHUNK

    ibufs, obufs = (ib0, ib1), (ob0, ob1)
    isems, osems = (si0, si1), (so0, so1)

    def start_in(k):
        return pltpu.async_copy(
            x_hbm.at[pl.ds(row0 + k * crows, crows)], ibufs[k % 2],
            isems[k % 2])

    def start_out(k):
        return pltpu.async_copy(
            obufs[k % 2], out_hbm.at[pl.ds(row0 + k * crows, crows)],
            osems[k % 2])

    in_cp = {0: start_in(0), 1: start_in(1)}
    pltpu.sync_copy(y_hbm, y_v)

    for j in range((n - 1) // _LANES):
        v = y_v[pl.ds(j * _LANES, _LANES)]
        vn = y_v[pl.ds(j * _LANES + 1, _LANES)]
        dy_v[pl.ds(j * _LANES, _LANES)] = vn - v

    scale = jnp.float32(n - 1)
    offs = list(range(0, cols - _LANES, _LANES)) + [cols - _LANES]
    out_cp = {}

    for k in range(_NCHUNK):
        ib, ob = ibufs[k % 2], obufs[k % 2]
        in_cp.pop(k).wait()
        if k >= 2:
            out_cp.pop(k - 2).wait()

        @plsc.parallel_loop(0, crows, unroll=4)
        def body(r, ib=ib, ob=ob):
            xs = [ib[r, pl.ds(c, _LANES)] for c in offs]
            for c, xv in zip(offs, xs):
                t = xv * scale
                i0 = t.astype(jnp.int32)
                w = t - i0.astype(jnp.float32)
                y0 = plsc.load_gather(y_v, [i0])
                d0 = plsc.load_gather(dy_v, [i0])
                ob[r, pl.ds(c, _LANES)] = y0 + w * d0
        out_cp[k] = start_out(k)
        if k + 2 < _NCHUNK:
            in_cp[k + 2] = start_in(k + 2)

    for k in sorted(out_cp):
        out_cp[k].wait()


def kernel(x, y):
    n = y.shape[0]
    rows, cols = x.shape
    nw = 32
    rows_per_w = rows // nw
    assert rows_per_w * nw == rows and cols >= _LANES
    assert (n - 1) % _LANES == 0 and rows_per_w % _NCHUNK == 0
    crows = rows_per_w // _NCHUNK

    mesh = plsc.VectorSubcoreMesh(core_axis_name="c", subcore_axis_name="s")
    run = pl.kernel(
        functools.partial(_sc_interp_kernel, rows_per_w, cols, n),
        mesh=mesh,
        out_type=jax.ShapeDtypeStruct((rows, cols), jnp.float32),
        scratch_types=[
            pltpu.VMEM((n,), jnp.float32),
            pltpu.VMEM((n - 1,), jnp.float32),
            pltpu.VMEM((crows, cols), jnp.float32),
            pltpu.VMEM((crows, cols), jnp.float32),
            pltpu.VMEM((crows, cols), jnp.float32),
            pltpu.VMEM((crows, cols), jnp.float32),
            pltpu.SemaphoreType.DMA,
            pltpu.SemaphoreType.DMA,
            pltpu.SemaphoreType.DMA,
            pltpu.SemaphoreType.DMA,
        ],
        compiler_params=pltpu.CompilerParams(
            needs_layout_passes=False, use_tc_tiling_on_sc=True),
    )
    return run(x, y)


# speedup vs baseline: 1.4520x; 1.4518x over previous
import functools

import jax
import jax.numpy as jnp
from jax import lax
from jax.experimental import pallas as pl
from jax.experimental.pallas import tpu as pltpu, tpu_sc as plsc

_LANES = 16
_NCHUNK = 4


def _sc_interp_kernel(rows, cols_per_w, n, x_hbm, y_hbm, out_hbm,
                      y_v, dy_v, ib0, ib1, ob0, ob1, si0, si1, so0, so1):
    wid = lax.axis_index("s") * 2 + lax.axis_index("c")
    col0 = wid * cols_per_w
    ccols = cols_per_w // _NCHUNK

    ibufs, obufs = (ib0, ib1), (ob0, ob1)
    isems, osems = (si0, si1), (so0, so1)

    def start_in(k):
        return pltpu.async_copy(
            x_hbm.at[:, pl.ds(col0 + k * ccols, ccols)], ibufs[k % 2],
            isems[k % 2])

    def start_out(k):
        return pltpu.async_copy(
            obufs[k % 2], out_hbm.at[:, pl.ds(col0 + k * ccols, ccols)],
            osems[k % 2])

    in_cp = {0: start_in(0), 1: start_in(1)}
    pltpu.sync_copy(y_hbm, y_v)

    for j in range((n - 1) // _LANES):
        v = y_v[pl.ds(j * _LANES, _LANES)]
        vn = y_v[pl.ds(j * _LANES + 1, _LANES)]
        dy_v[pl.ds(j * _LANES, _LANES)] = vn - v

    scale = jnp.float32(n - 1)
    out_cp = {}

    for k in range(_NCHUNK):
        ib, ob = ibufs[k % 2], obufs[k % 2]
        in_cp.pop(k).wait()
        if k >= 2:
            out_cp.pop(k - 2).wait()

        @plsc.parallel_loop(0, rows, unroll=2)
        def body(r, ib=ib, ob=ob):
            xs = [ib[r, pl.ds(c, _LANES)] for c in range(0, ccols, _LANES)]
            for c, xv in zip(range(0, ccols, _LANES), xs):
                t = xv * scale
                i0 = t.astype(jnp.int32)
                w = t - i0.astype(jnp.float32)
                y0 = plsc.load_gather(y_v, [i0])
                d0 = plsc.load_gather(dy_v, [i0])
                ob[r, pl.ds(c, _LANES)] = y0 + w * d0

        out_cp[k] = start_out(k)
        if k + 2 < _NCHUNK:
            in_cp[k + 2] = start_in(k + 2)

    for k in sorted(out_cp):
        out_cp[k].wait()


def kernel(x, y):
    n = y.shape[0]
    xt = x.T
    rows, cols = xt.shape
    nw = 32
    cols_per_w = cols // nw
    assert cols_per_w * nw == cols
    assert (n - 1) % _LANES == 0 and cols_per_w % (_NCHUNK * _LANES) == 0
    ccols = cols_per_w // _NCHUNK

    mesh = plsc.VectorSubcoreMesh(core_axis_name="c", subcore_axis_name="s")
    run = pl.kernel(
        functools.partial(_sc_interp_kernel, rows, cols_per_w, n),
        mesh=mesh,
        out_type=jax.ShapeDtypeStruct((rows, cols), jnp.float32),
        scratch_types=[
            pltpu.VMEM((n,), jnp.float32),
            pltpu.VMEM((n - 1,), jnp.float32),
            pltpu.VMEM((rows, ccols), jnp.float32),
            pltpu.VMEM((rows, ccols), jnp.float32),
            pltpu.VMEM((rows, ccols), jnp.float32),
            pltpu.VMEM((rows, ccols), jnp.float32),
            pltpu.SemaphoreType.DMA,
            pltpu.SemaphoreType.DMA,
            pltpu.SemaphoreType.DMA,
            pltpu.SemaphoreType.DMA,
        ],
        compiler_params=pltpu.CompilerParams(needs_layout_passes=False),
    )
    return run(xt, y).T


# speedup vs baseline: 1.4982x; 1.0318x over previous
import functools

import jax
import jax.numpy as jnp
from jax import lax
from jax.experimental import pallas as pl
from jax.experimental.pallas import tpu as pltpu, tpu_sc as plsc

_LANES = 16
_NCHUNK = 4


def _sc_interp_kernel(rows, cols_per_w, n, x_hbm, y_hbm, out_hbm,
                      y_v, pk_v, ib0, ib1, ob0, ob1, si0, si1, so0, so1):
    wid = lax.axis_index("s") * 2 + lax.axis_index("c")
    col0 = wid * cols_per_w
    ccols = cols_per_w // _NCHUNK

    ibufs, obufs = (ib0, ib1), (ob0, ob1)
    isems, osems = (si0, si1), (so0, so1)

    def start_in(k):
        return pltpu.async_copy(
            x_hbm.at[:, pl.ds(col0 + k * ccols, ccols)], ibufs[k % 2],
            isems[k % 2])

    def start_out(k):
        return pltpu.async_copy(
            obufs[k % 2], out_hbm.at[:, pl.ds(col0 + k * ccols, ccols)],
            osems[k % 2])

    in_cp = {0: start_in(0), 1: start_in(1)}
    pltpu.sync_copy(y_hbm, y_v)

    def _rne_hi(f):
        b = plsc.bitcast(f, jnp.int32)
        rnd = jnp.int32(0x7FFF) + ((b >> 16) & jnp.int32(1))
        return (b + rnd) & jnp.int32(-65536)

    for j in range((n - 1) // _LANES):
        v = y_v[pl.ds(j * _LANES, _LANES)]
        vn = y_v[pl.ds(j * _LANES + 1, _LANES)]
        hi = _rne_hi(v)
        lo = lax.shift_right_logical(_rne_hi(vn - v), jnp.int32(16))
        pk_v[pl.ds(j * _LANES, _LANES)] = hi | lo

    scale = jnp.float32(n - 1)
    out_cp = {}

    for k in range(_NCHUNK):
        ib, ob = ibufs[k % 2], obufs[k % 2]
        in_cp.pop(k).wait()
        if k >= 2:
            out_cp.pop(k - 2).wait()

        @plsc.parallel_loop(0, rows, unroll=2)
        def body(r, ib=ib, ob=ob):
            xs = [ib[r, pl.ds(c, _LANES)] for c in range(0, ccols, _LANES)]
            for c, xv in zip(range(0, ccols, _LANES), xs):
                t = xv * scale
                i0 = t.astype(jnp.int32)
                w = t - i0.astype(jnp.float32)
                g = plsc.load_gather(pk_v, [i0])
                y0 = plsc.bitcast(g & jnp.int32(-65536), jnp.float32)
                d0 = plsc.bitcast(
                    lax.shift_left(g, jnp.int32(16)), jnp.float32)
                ob[r, pl.ds(c, _LANES)] = y0 + w * d0

        out_cp[k] = start_out(k)
        if k + 2 < _NCHUNK:
            in_cp[k + 2] = start_in(k + 2)

    for k in sorted(out_cp):
        out_cp[k].wait()


def kernel(x, y):
    n = y.shape[0]
    xt = x.T
    rows, cols = xt.shape
    nw = 32
    cols_per_w = cols // nw
    assert cols_per_w * nw == cols
    assert (n - 1) % _LANES == 0 and cols_per_w % (_NCHUNK * _LANES) == 0
    ccols = cols_per_w // _NCHUNK

    mesh = plsc.VectorSubcoreMesh(core_axis_name="c", subcore_axis_name="s")
    run = pl.kernel(
        functools.partial(_sc_interp_kernel, rows, cols_per_w, n),
        mesh=mesh,
        out_type=jax.ShapeDtypeStruct((rows, cols), jnp.float32),
        scratch_types=[
            pltpu.VMEM((n,), jnp.float32),
            pltpu.VMEM((n - 1,), jnp.int32),
            pltpu.VMEM((rows, ccols), jnp.float32),
            pltpu.VMEM((rows, ccols), jnp.float32),
            pltpu.VMEM((rows, ccols), jnp.float32),
            pltpu.VMEM((rows, ccols), jnp.float32),
            pltpu.SemaphoreType.DMA,
            pltpu.SemaphoreType.DMA,
            pltpu.SemaphoreType.DMA,
            pltpu.SemaphoreType.DMA,
        ],
        compiler_params=pltpu.CompilerParams(needs_layout_passes=False),
    )
    return run(xt, y).T


# speedup vs baseline: 1.4983x; 1.0001x over previous
import functools

import jax
import jax.numpy as jnp
from jax import lax
from jax.experimental import pallas as pl
from jax.experimental.pallas import tpu as pltpu, tpu_sc as plsc

_LANES = 16
_NCHUNK = 4


def _sc_interp_kernel(rows, cols_per_w, n, x_hbm, y_hbm, out_hbm,
                      y_v, pk_v, ib0, ib1, ob0, ob1, si0, si1, so0, so1):
    wid = lax.axis_index("s") * 2 + lax.axis_index("c")
    col0 = wid * cols_per_w
    ccols = cols_per_w // _NCHUNK

    ibufs, obufs = (ib0, ib1), (ob0, ob1)
    isems, osems = (si0, si1), (so0, so1)

    def start_in(k):
        return pltpu.async_copy(
            x_hbm.at[:, pl.ds(col0 + k * ccols, ccols)], ibufs[k % 2],
            isems[k % 2])

    def start_out(k):
        return pltpu.async_copy(
            obufs[k % 2], out_hbm.at[:, pl.ds(col0 + k * ccols, ccols)],
            osems[k % 2])

    in_cp = {0: start_in(0), 1: start_in(1)}
    pltpu.sync_copy(y_hbm, y_v)

    def _rne_hi(f):
        b = plsc.bitcast(f, jnp.int32)
        rnd = jnp.int32(0x7FFF) + ((b >> 16) & jnp.int32(1))
        return (b + rnd) & jnp.int32(-65536)

    for j in range((n - 1) // _LANES):
        v = y_v[pl.ds(j * _LANES, _LANES)]
        vn = y_v[pl.ds(j * _LANES + 1, _LANES)]
        hi = _rne_hi(v)
        lo = lax.shift_right_logical(_rne_hi(vn - v), jnp.int32(16))
        pk_v[pl.ds(j * _LANES, _LANES)] = hi | lo

    scale = jnp.float32(n - 1)
    out_cp = {}

    for k in range(_NCHUNK):
        ib, ob = ibufs[k % 2], obufs[k % 2]
        in_cp.pop(k).wait()
        if k >= 2:
            out_cp.pop(k - 2).wait()

        @plsc.parallel_loop(0, rows, unroll=4)
        def body(r, ib=ib, ob=ob):
            xs = [ib[r, pl.ds(c, _LANES)] for c in range(0, ccols, _LANES)]
            for c, xv in zip(range(0, ccols, _LANES), xs):
                t = xv * scale
                i0 = t.astype(jnp.int32)
                w = t - i0.astype(jnp.float32)
                g = plsc.load_gather(pk_v, [i0])
                y0 = plsc.bitcast(g & jnp.int32(-65536), jnp.float32)
                d0 = plsc.bitcast(
                    lax.shift_left(g, jnp.int32(16)), jnp.float32)
                ob[r, pl.ds(c, _LANES)] = y0 + w * d0

        out_cp[k] = start_out(k)
        if k + 2 < _NCHUNK:
            in_cp[k + 2] = start_in(k + 2)

    for k in sorted(out_cp):
        out_cp[k].wait()


def kernel(x, y):
    n = y.shape[0]
    xt = x.T
    rows, cols = xt.shape
    nw = 32
    cols_per_w = cols // nw
    assert cols_per_w * nw == cols
    assert (n - 1) % _LANES == 0 and cols_per_w % (_NCHUNK * _LANES) == 0
    ccols = cols_per_w // _NCHUNK

    mesh = plsc.VectorSubcoreMesh(core_axis_name="c", subcore_axis_name="s")
    run = pl.kernel(
        functools.partial(_sc_interp_kernel, rows, cols_per_w, n),
        mesh=mesh,
        out_type=jax.ShapeDtypeStruct((rows, cols), jnp.float32),
        scratch_types=[
            pltpu.VMEM((n,), jnp.float32),
            pltpu.VMEM((n - 1,), jnp.int32),
            pltpu.VMEM((rows, ccols), jnp.float32),
            pltpu.VMEM((rows, ccols), jnp.float32),
            pltpu.VMEM((rows, ccols), jnp.float32),
            pltpu.VMEM((rows, ccols), jnp.float32),
            pltpu.SemaphoreType.DMA,
            pltpu.SemaphoreType.DMA,
            pltpu.SemaphoreType.DMA,
            pltpu.SemaphoreType.DMA,
        ],
        compiler_params=pltpu.CompilerParams(needs_layout_passes=False),
    )
    return run(xt, y).T
